# Initial kernel scaffold; baseline (speedup 1.0000x reference)
import jax
import jax.numpy as jnp
from jax.experimental import pallas as pl


def kernel(x, table):
    raise NotImplementedError("write your pallas kernel here")


# speedup vs baseline: 1.3082x; 1.3082x over previous
import functools

import jax
import jax.numpy as jnp
from jax import lax
from jax.experimental import pallas as pl
from jax.experimental.pallas import tpu as pltpu
from jax.experimental.pallas import tpu_sc as plsc

NTOKEN = 1000000
EMB_DIM = 32
BATCH = 16384
HIST = 50

B = BATCH * HIST
NC, NS = 2, 16
NW = NC * NS
BPW = B // NW
ROW = 128
NROWS = BPW // ROW
K = 20
NCHUNK = NROWS // K

_mesh = plsc.VectorSubcoreMesh(core_axis_name="c", subcore_axis_name="s")


@functools.partial(
    pl.kernel,
    mesh=_mesh,
    out_type=jax.ShapeDtypeStruct((B // ROW, ROW, EMB_DIM), jnp.float32),
    scratch_types=[
        pltpu.VMEM((NROWS, ROW), jnp.int32),
        pltpu.VMEM((K, ROW, EMB_DIM), jnp.float32),
        pltpu.SemaphoreType.DMA,
    ],
    compiler_params=pltpu.CompilerParams(use_tc_tiling_on_sc=False),
)
def _gather_kernel(idx_hbm, table_hbm, out_hbm, idx_v, rows_v, sem):
    wid = lax.axis_index("s") * NC + lax.axis_index("c")
    base = wid * NROWS
    pltpu.sync_copy(idx_hbm.at[pl.ds(base, NROWS)], idx_v)

    def chunk(c, carry):
        r0 = c * K
        copies = [
            pltpu.async_copy(
                table_hbm.at[idx_v.at[r0 + j]], rows_v.at[j], sem)
            for j in range(K)
        ]
        for cp in copies:
            cp.wait()
        pltpu.sync_copy(rows_v, out_hbm.at[pl.ds(base + r0, K)])
        return carry

    lax.fori_loop(0, NCHUNK, chunk, 0)


def kernel(x, table):
    idx = x.reshape(B // ROW, ROW).astype(jnp.int32)
    out = _gather_kernel(idx, table)
    return out.reshape(BATCH, HIST, EMB_DIM)
